# baseline (speedup 1.0000x reference)
import jax
import jax.numpy as jnp
from jax.experimental import pallas as pl
from jax.experimental.pallas import tpu as pltpu

D_MODEL = 2048
NUM_EXPERTS = 64
N_TOKENS = 8192
CONFIDENCE_THRESHOLD = 0.5
TOKEN_TILE = 2048
PAIR_ROWS = TOKEN_TILE // 2
LANES = 2 * NUM_EXPERTS


def _lane_group_iota():
    return jax.lax.broadcasted_iota(jnp.int32, (1, LANES), 1) & (NUM_EXPERTS - 1)


def _sort_desc_groups(v):
    idx = _lane_group_iota()
    k = 2
    while k <= NUM_EXPERTS:
        d = (idx & k) != 0
        j = k // 2
        while j >= 1:
            m = (idx & j) != 0
            pv = jnp.where(m, pltpu.roll(v, j, 1), pltpu.roll(v, LANES - j, 1))
            v = jnp.where(m == d, jnp.maximum(v, pv), jnp.minimum(v, pv))
            j //= 2
        k *= 2
    return v


def _cumsum_groups(v):
    idx = _lane_group_iota()
    s = 1
    while s < NUM_EXPERTS:
        v = v + jnp.where(idx >= s, pltpu.roll(v, s, 1), 0.0)
        s *= 2
    return v


def _butterfly(v, combine):
    idx = _lane_group_iota()
    s = 1
    while s < NUM_EXPERTS:
        pv = jnp.where((idx & s) != 0,
                       pltpu.roll(v, s, 1), pltpu.roll(v, LANES - s, 1))
        v = combine(v, pv)
        s *= 2
    return v


def _router_kernel(x_ref, w_ref, b_ref, rw_ref, probs_ref, cnt_ref):
    lg = jnp.dot(x_ref[...], w_ref[...], preferred_element_type=jnp.float32,
                 precision=jax.lax.Precision.DEFAULT) + b_ref[...]
    logits = jnp.concatenate(
        [jax.lax.slice(lg, (0, 0), (PAIR_ROWS, NUM_EXPERTS)),
         jax.lax.slice(lg, (PAIR_ROWS, 0), (TOKEN_TILE, NUM_EXPERTS))],
        axis=1)

    gi = jax.lax.broadcasted_iota(jnp.int32, (LANES, LANES), 0)
    gj = jax.lax.broadcasted_iota(jnp.int32, (LANES, LANES), 1)
    bd = ((gi // NUM_EXPERTS) == (gj // NUM_EXPERTS)).astype(jnp.float32)

    def gsum(a):
        return jnp.dot(a, bd, preferred_element_type=jnp.float32,
                       precision=jax.lax.Precision.HIGHEST)

    mx = _butterfly(logits, jnp.maximum)
    ex = jnp.exp(logits - mx)
    p = ex / gsum(ex)

    sp = _sort_desc_groups(p)
    shifted = _cumsum_groups(sp) - sp
    act_s = shifted < CONFIDENCE_THRESHOLD
    act_p_s = jnp.where(act_s, sp, 0.0)
    mass = gsum(act_p_s)
    theta = _butterfly(jnp.where(act_s, sp, jnp.inf), jnp.minimum)
    r = gsum((act_s & (sp == theta)).astype(jnp.float32))

    eqf = (p == theta).astype(jnp.float32)
    rank_excl = _cumsum_groups(eqf) - eqf
    active = (p > theta) | ((p == theta) & (rank_excl < r))

    def unpack(a):
        return jnp.concatenate(
            [jax.lax.slice(a, (0, 0), (PAIR_ROWS, NUM_EXPERTS)),
             jax.lax.slice(a, (0, NUM_EXPERTS), (PAIR_ROWS, LANES))],
            axis=0)

    active_probs = jnp.where(active, p, 0.0)
    rw_ref[...] = unpack(active_probs / (mass + 1e-6))
    probs_ref[...] = unpack(p)

    sel = (jax.lax.broadcasted_iota(jnp.int32, (LANES, 2), 0) // NUM_EXPERTS
           == jax.lax.broadcasted_iota(jnp.int32, (LANES, 2), 1)
           ).astype(jnp.float32)
    c2 = jnp.dot(act_s.astype(jnp.float32), sel,
                 preferred_element_type=jnp.float32,
                 precision=jax.lax.Precision.HIGHEST).astype(jnp.int32)
    cnt_ref[...] = jnp.concatenate(
        [jax.lax.slice(c2, (0, 0), (PAIR_ROWS, 1)),
         jax.lax.slice(c2, (0, 1), (PAIR_ROWS, 2))], axis=0)


def kernel(x, W, b):
    n_tiles = N_TOKENS // TOKEN_TILE
    b2 = b.reshape(1, NUM_EXPERTS)
    rw, probs, cnt = pl.pallas_call(
        _router_kernel,
        grid=(n_tiles,),
        compiler_params=pltpu.CompilerParams(
            dimension_semantics=("parallel",)),
        in_specs=[
            pl.BlockSpec((TOKEN_TILE, D_MODEL), lambda i: (i, 0)),
            pl.BlockSpec((D_MODEL, NUM_EXPERTS), lambda i: (0, 0)),
            pl.BlockSpec((1, NUM_EXPERTS), lambda i: (0, 0)),
        ],
        out_specs=[
            pl.BlockSpec((TOKEN_TILE, NUM_EXPERTS), lambda i: (i, 0)),
            pl.BlockSpec((TOKEN_TILE, NUM_EXPERTS), lambda i: (i, 0)),
            pl.BlockSpec((TOKEN_TILE, 1), lambda i: (i, 0)),
        ],
        out_shape=[
            jax.ShapeDtypeStruct((N_TOKENS, NUM_EXPERTS), jnp.float32),
            jax.ShapeDtypeStruct((N_TOKENS, NUM_EXPERTS), jnp.float32),
            jax.ShapeDtypeStruct((N_TOKENS, 1), jnp.int32),
        ],
    )(x, W, b2)
    return rw, probs, cnt.reshape(N_TOKENS)


# speedup vs baseline: 1.0328x; 1.0328x over previous
import jax
import jax.numpy as jnp
from jax.experimental import pallas as pl
from jax.experimental.pallas import tpu as pltpu

D_MODEL = 2048
NUM_EXPERTS = 64
N_TOKENS = 8192
CONFIDENCE_THRESHOLD = 0.5
TOKEN_TILE = 2048
PAIR_ROWS = TOKEN_TILE // 2
LANES = 2 * NUM_EXPERTS


def _lane_group_iota():
    return jax.lax.broadcasted_iota(jnp.int32, (1, LANES), 1) & (NUM_EXPERTS - 1)


def _sort_desc_groups(v):
    idx = _lane_group_iota()
    k = 2
    while k <= NUM_EXPERTS:
        d = (idx & k) != 0
        j = k // 2
        while j >= 1:
            m = (idx & j) != 0
            pv = jnp.where(m, pltpu.roll(v, j, 1), pltpu.roll(v, LANES - j, 1))
            v = jnp.where(m == d, jnp.maximum(v, pv), jnp.minimum(v, pv))
            j //= 2
        k *= 2
    return v


def _cumsum_groups(v):
    idx = _lane_group_iota()
    s = 1
    while s < NUM_EXPERTS:
        v = v + jnp.where(idx >= s, pltpu.roll(v, s, 1), 0.0)
        s *= 2
    return v


def _butterfly(v, combine):
    idx = _lane_group_iota()
    s = 1
    while s < NUM_EXPERTS:
        pv = jnp.where((idx & s) != 0,
                       pltpu.roll(v, s, 1), pltpu.roll(v, LANES - s, 1))
        v = combine(v, pv)
        s *= 2
    return v


def _router_kernel(x_ref, w_ref, b_ref, rw_ref, probs_ref, cnt_ref):
    lg = jnp.dot(x_ref[...], w_ref[...], preferred_element_type=jnp.float32,
                 precision=jax.lax.Precision.DEFAULT) + b_ref[...]
    logits = jnp.concatenate(
        [jax.lax.slice(lg, (0, 0), (PAIR_ROWS, NUM_EXPERTS)),
         jax.lax.slice(lg, (PAIR_ROWS, 0), (TOKEN_TILE, NUM_EXPERTS))],
        axis=1)

    gi = jax.lax.broadcasted_iota(jnp.int32, (LANES, LANES), 0)
    gj = jax.lax.broadcasted_iota(jnp.int32, (LANES, LANES), 1)
    bd = ((gi // NUM_EXPERTS) == (gj // NUM_EXPERTS)).astype(jnp.float32)

    def gsum(a):
        return jnp.dot(a, bd, preferred_element_type=jnp.float32,
                       precision=jax.lax.Precision.HIGHEST)

    mx = jnp.max(logits, axis=-1, keepdims=True)
    ex = jnp.exp(logits - mx)
    p = ex / gsum(ex)

    sp = _sort_desc_groups(p)
    cum = _cumsum_groups(sp)
    shifted = cum - sp
    act_s = shifted < CONFIDENCE_THRESHOLD
    act_p_s = jnp.where(act_s, sp, 0.0)
    mass = gsum(act_p_s)
    idx = _lane_group_iota()
    boundary = act_s & ((cum >= CONFIDENCE_THRESHOLD)
                        | (idx == NUM_EXPERTS - 1))
    theta = gsum(jnp.where(boundary, sp, 0.0))
    r = gsum((act_s & (sp == theta)).astype(jnp.float32))

    eqf = (p == theta).astype(jnp.float32)
    rank_excl = _cumsum_groups(eqf) - eqf
    active = (p > theta) | ((p == theta) & (rank_excl < r))

    def unpack(a):
        return jnp.concatenate(
            [jax.lax.slice(a, (0, 0), (PAIR_ROWS, NUM_EXPERTS)),
             jax.lax.slice(a, (0, NUM_EXPERTS), (PAIR_ROWS, LANES))],
            axis=0)

    active_probs = jnp.where(active, p, 0.0)
    rw_ref[...] = unpack(active_probs / (mass + 1e-6))
    probs_ref[...] = unpack(p)

    sel = (jax.lax.broadcasted_iota(jnp.int32, (LANES, 2), 0) // NUM_EXPERTS
           == jax.lax.broadcasted_iota(jnp.int32, (LANES, 2), 1)
           ).astype(jnp.float32)
    c2 = jnp.dot(act_s.astype(jnp.float32), sel,
                 preferred_element_type=jnp.float32,
                 precision=jax.lax.Precision.HIGHEST).astype(jnp.int32)
    cnt_ref[...] = jnp.concatenate(
        [jax.lax.slice(c2, (0, 0), (PAIR_ROWS, 1)),
         jax.lax.slice(c2, (0, 1), (PAIR_ROWS, 2))], axis=0)


def kernel(x, W, b):
    n_tiles = N_TOKENS // TOKEN_TILE
    b2 = b.reshape(1, NUM_EXPERTS)
    rw, probs, cnt = pl.pallas_call(
        _router_kernel,
        grid=(n_tiles,),
        compiler_params=pltpu.CompilerParams(
            dimension_semantics=("parallel",)),
        in_specs=[
            pl.BlockSpec((TOKEN_TILE, D_MODEL), lambda i: (i, 0)),
            pl.BlockSpec((D_MODEL, NUM_EXPERTS), lambda i: (0, 0)),
            pl.BlockSpec((1, NUM_EXPERTS), lambda i: (0, 0)),
        ],
        out_specs=[
            pl.BlockSpec((TOKEN_TILE, NUM_EXPERTS), lambda i: (i, 0)),
            pl.BlockSpec((TOKEN_TILE, NUM_EXPERTS), lambda i: (i, 0)),
            pl.BlockSpec((TOKEN_TILE, 1), lambda i: (i, 0)),
        ],
        out_shape=[
            jax.ShapeDtypeStruct((N_TOKENS, NUM_EXPERTS), jnp.float32),
            jax.ShapeDtypeStruct((N_TOKENS, NUM_EXPERTS), jnp.float32),
            jax.ShapeDtypeStruct((N_TOKENS, 1), jnp.int32),
        ],
    )(x, W, b2)
    return rw, probs, cnt.reshape(N_TOKENS)
